# Initial kernel scaffold; baseline (speedup 1.0000x reference)
import jax
import jax.numpy as jnp
from jax.experimental import pallas as pl


def kernel(atomic_numbers, embedding_weight):
    raise NotImplementedError("write your pallas kernel here")


# speedup vs baseline: 1.4379x; 1.4379x over previous
import jax
import jax.numpy as jnp
from jax import lax
from jax.experimental import pallas as pl
from jax.experimental.pallas import tpu as pltpu
from jax.experimental.pallas import tpu_sc as plsc

_N = 100000
_DIM = 128
_C = 128
_NFULL = _N // _C
_TAIL = _N - _NFULL * _C
_NCHUNK = _NFULL + 1
_NBUF = 2

_info = plsc.get_sparse_core_info()
_NCORES = _info.num_cores
_NSUB = _info.num_subcores
_NW = _NCORES * _NSUB
_MAXT = -(-_NCHUNK // _NW)
_TLOOP = -(-_MAXT // _NBUF) * _NBUF


def _body(idx_hbm, table_hbm, out_hbm, idx_v, rows_v, sem0, sem1):
    wid = lax.axis_index("s") * _NCORES + lax.axis_index("c")
    sems = [sem0, sem1]

    def start(t, b):
        cid = wid + t * _NW

        @pl.when(cid < _NFULL)
        def _():
            pltpu.sync_copy(idx_hbm.at[pl.ds(cid * _C, _C)], idx_v.at[b])
            pltpu.async_copy(table_hbm.at[idx_v.at[b]], rows_v.at[b], sems[b])

        @pl.when(cid == _NFULL)
        def _():
            pltpu.sync_copy(
                idx_hbm.at[pl.ds(_NFULL * _C, _TAIL)],
                idx_v.at[b].at[pl.ds(0, _TAIL)],
            )
            pltpu.async_copy(
                table_hbm.at[idx_v.at[b].at[pl.ds(0, _TAIL)]],
                rows_v.at[b].at[pl.ds(0, _TAIL)],
                sems[b],
            )

    def finish(t, b):
        cid = wid + t * _NW

        @pl.when(cid < _NFULL)
        def _():
            pltpu.make_async_copy(
                table_hbm.at[idx_v.at[b]], rows_v.at[b], sems[b]
            ).wait()
            pltpu.sync_copy(rows_v.at[b], out_hbm.at[pl.ds(cid * _C, _C)])

        @pl.when(cid == _NFULL)
        def _():
            pltpu.make_async_copy(
                table_hbm.at[idx_v.at[b].at[pl.ds(0, _TAIL)]],
                rows_v.at[b].at[pl.ds(0, _TAIL)],
                sems[b],
            ).wait()
            pltpu.sync_copy(
                rows_v.at[b].at[pl.ds(0, _TAIL)],
                out_hbm.at[pl.ds(_NFULL * _C, _TAIL)],
            )

    for b in range(_NBUF):
        start(b, b)

    @pl.loop(0, _TLOOP, step=_NBUF)
    def _(g):
        for b in range(_NBUF):
            t = g + b
            finish(t, b)
            start(t + _NBUF, b)


def kernel(atomic_numbers, embedding_weight):
    idx = atomic_numbers.astype(jnp.int32)
    run = pl.kernel(
        _body,
        out_type=jax.ShapeDtypeStruct((_N, _DIM), jnp.float32),
        mesh=plsc.VectorSubcoreMesh(core_axis_name="c", subcore_axis_name="s"),
        scratch_types=[
            pltpu.VMEM((_NBUF, _C), jnp.int32),
            pltpu.VMEM((_NBUF, _C, _DIM), jnp.float32),
            pltpu.SemaphoreType.DMA,
            pltpu.SemaphoreType.DMA,
        ],
    )
    return run(idx, embedding_weight)
